# baseline (speedup 1.0000x reference)
import functools

import jax
import jax.numpy as jnp
from jax import lax
from jax.experimental import pallas as pl
from jax.experimental.pallas import tpu as pltpu
from jax.experimental.pallas import tpu_sc as plsc

BATCH = 16384
HIDDEN = 64
L = 16

_info = plsc.get_sparse_core_info()
NC, NS = _info.num_cores, _info.num_subcores
NW = NC * NS
BPW = BATCH // NW
CHUNK = 128
NCHUNK = BPW // CHUNK
GROUPS = BPW // L

_mesh = plsc.VectorSubcoreMesh(core_axis_name="c", subcore_axis_name="s")


@functools.partial(
    pl.kernel,
    mesh=_mesh,
    out_type=jax.ShapeDtypeStruct((BATCH,), jnp.float32),
    compiler_params=pltpu.CompilerParams(
        needs_layout_passes=False, use_tc_tiling_on_sc=False
    ),
    scratch_types=[
        pltpu.VMEM((NCHUNK, CHUNK), jnp.int32),
        pltpu.VMEM((NCHUNK, CHUNK), jnp.int32),
        pltpu.VMEM((NCHUNK, CHUNK), jnp.int32),
        pltpu.VMEM((BPW, HIDDEN), jnp.float32),
        pltpu.VMEM((BPW, HIDDEN), jnp.float32),
        pltpu.VMEM((BPW, HIDDEN), jnp.float32),
        pltpu.VMEM((BPW,), jnp.float32),
        pltpu.SemaphoreType.DMA,
    ],
)
def _kge_score_sc(head_hbm, rel_hbm, tail_hbm, node_hbm, relemb_hbm, out_hbm,
                  idx_h, idx_r, idx_t, h_rows, r_rows, t_rows, scores, sem):
    wid = lax.axis_index("s") * NC + lax.axis_index("c")
    base = wid * BPW

    for c in range(NCHUNK):
        off = base + c * CHUNK
        pltpu.sync_copy(head_hbm.at[pl.ds(off, CHUNK)], idx_h.at[c])
        pltpu.sync_copy(rel_hbm.at[pl.ds(off, CHUNK)], idx_r.at[c])
        pltpu.sync_copy(tail_hbm.at[pl.ds(off, CHUNK)], idx_t.at[c])

    copies = []
    for c in range(NCHUNK):
        dst = pl.ds(c * CHUNK, CHUNK)
        copies.append(pltpu.async_copy(node_hbm.at[idx_h.at[c]], h_rows.at[dst], sem))
        copies.append(pltpu.async_copy(relemb_hbm.at[idx_r.at[c]], r_rows.at[dst], sem))
        copies.append(pltpu.async_copy(node_hbm.at[idx_t.at[c]], t_rows.at[dst], sem))
    for cp in copies:
        cp.wait()

    lanes = lax.iota(jnp.int32, L)

    def group_body(g, carry):
        row_idx = g * L + lanes

        def col_body(j, acc):
            col_idx = jnp.full((L,), j, dtype=jnp.int32)
            h = plsc.load_gather(h_rows, [row_idx, col_idx])
            r = plsc.load_gather(r_rows, [row_idx, col_idx])
            t = plsc.load_gather(t_rows, [row_idx, col_idx])
            return acc + jnp.abs(h + r - t)

        acc = lax.fori_loop(0, HIDDEN, col_body, jnp.zeros((L,), jnp.float32))
        scores[pl.ds(pl.multiple_of(g * L, L), L)] = -acc
        return carry

    lax.fori_loop(0, GROUPS, group_body, 0)
    pltpu.sync_copy(scores, out_hbm.at[pl.ds(base, BPW)])


def kernel(head_index, rel_type, tail_index, node_emb, rel_emb):
    return _kge_score_sc(
        head_index.astype(jnp.int32),
        rel_type.astype(jnp.int32),
        tail_index.astype(jnp.int32),
        node_emb,
        rel_emb,
    )


# speedup vs baseline: 1.5493x; 1.5493x over previous
import functools

import jax
import jax.numpy as jnp
from jax import lax
from jax.experimental import pallas as pl
from jax.experimental.pallas import tpu as pltpu
from jax.experimental.pallas import tpu_sc as plsc

BATCH = 16384
HIDDEN = 64
L = 16

_info = plsc.get_sparse_core_info()
NC, NS = _info.num_cores, _info.num_subcores
NW = NC * NS
BPW = BATCH // NW
CH = 16
NCHUNK = BPW // CH

_mesh = plsc.VectorSubcoreMesh(core_axis_name="c", subcore_axis_name="s")


@functools.partial(
    pl.kernel,
    mesh=_mesh,
    out_type=jax.ShapeDtypeStruct((BATCH,), jnp.float32),
    compiler_params=pltpu.CompilerParams(needs_layout_passes=False),
    scratch_types=[
        pltpu.VMEM((BPW,), jnp.int32),
        pltpu.VMEM((BPW,), jnp.int32),
        pltpu.VMEM((BPW,), jnp.int32),
        pltpu.VMEM((CH, HIDDEN), jnp.float32),
        pltpu.VMEM((CH, HIDDEN), jnp.float32),
        pltpu.VMEM((CH, HIDDEN), jnp.float32),
        pltpu.VMEM((BPW,), jnp.float32),
        pltpu.SemaphoreType.DMA,
    ],
)
def _kge_score_sc(head_hbm, rel_hbm, tail_hbm, node_hbm, relemb_hbm, out_hbm,
                  idx_h, idx_r, idx_t, hbuf, rbuf, tbuf, scores, sem):
    wid = lax.axis_index("s") * NC + lax.axis_index("c")
    base = wid * BPW

    pltpu.sync_copy(head_hbm.at[pl.ds(base, BPW)], idx_h)
    pltpu.sync_copy(rel_hbm.at[pl.ds(base, BPW)], idx_r)
    pltpu.sync_copy(tail_hbm.at[pl.ds(base, BPW)], idx_t)

    lanes = lax.iota(jnp.int32, L)

    def chunk_body(c, carry):
        off = pl.multiple_of(c * CH, CH)
        ihv = idx_h[pl.ds(off, CH)]
        irv = idx_r[pl.ds(off, CH)]
        itv = idx_t[pl.ds(off, CH)]
        copies = []
        for k in range(CH):
            dst = pl.ds(k, 1)
            copies.append(pltpu.async_copy(
                node_hbm.at[pl.ds(ihv[k], 1), :], hbuf.at[dst], sem))
            copies.append(pltpu.async_copy(
                relemb_hbm.at[pl.ds(irv[k], 1), :], rbuf.at[dst], sem))
            copies.append(pltpu.async_copy(
                node_hbm.at[pl.ds(itv[k], 1), :], tbuf.at[dst], sem))
        for cp in copies:
            cp.wait()

        def col_body(j, acc):
            cj = jnp.full((L,), j, dtype=jnp.int32)
            h = plsc.load_gather(hbuf, [lanes, cj])
            r = plsc.load_gather(rbuf, [lanes, cj])
            t = plsc.load_gather(tbuf, [lanes, cj])
            return acc + jnp.abs(h + r - t)

        acc = lax.fori_loop(0, HIDDEN, col_body, jnp.zeros((L,), jnp.float32))
        scores[pl.ds(off, CH)] = -acc
        return carry

    lax.fori_loop(0, NCHUNK, chunk_body, 0)
    pltpu.sync_copy(scores, out_hbm.at[pl.ds(base, BPW)])


def kernel(head_index, rel_type, tail_index, node_emb, rel_emb):
    return _kge_score_sc(
        head_index.astype(jnp.int32),
        rel_type.astype(jnp.int32),
        tail_index.astype(jnp.int32),
        node_emb,
        rel_emb,
    )


# speedup vs baseline: 1.6326x; 1.0538x over previous
import functools

import jax
import jax.numpy as jnp
from jax import lax
from jax.experimental import pallas as pl
from jax.experimental.pallas import tpu as pltpu
from jax.experimental.pallas import tpu_sc as plsc

BATCH = 16384
HIDDEN = 64
L = 16

_info = plsc.get_sparse_core_info()
NC, NS = _info.num_cores, _info.num_subcores
NW = NC * NS
BPW = BATCH // NW
CH = 16
NCHUNK = BPW // CH
NBUF = 4

_mesh = plsc.VectorSubcoreMesh(core_axis_name="c", subcore_axis_name="s")

_row_bufs = [pltpu.VMEM((CH, HIDDEN), jnp.float32)
             for _ in range(3 * NBUF)]


@functools.partial(
    pl.kernel,
    mesh=_mesh,
    out_type=jax.ShapeDtypeStruct((BATCH,), jnp.float32),
    compiler_params=pltpu.CompilerParams(needs_layout_passes=False),
    scratch_types=[
        pltpu.VMEM((BPW,), jnp.int32),
        pltpu.VMEM((BPW,), jnp.int32),
        pltpu.VMEM((BPW,), jnp.int32),
        pltpu.VMEM((BPW,), jnp.float32),
    ] + _row_bufs + [pltpu.SemaphoreType.DMA for _ in range(NBUF)],
)
def _kge_score_sc(head_hbm, rel_hbm, tail_hbm, node_hbm, relemb_hbm, out_hbm,
                  idx_h, idx_r, idx_t, scores, *bufs_and_sems):
    bufs = [bufs_and_sems[3 * b:3 * b + 3] for b in range(NBUF)]
    sems = bufs_and_sems[3 * NBUF:]

    wid = lax.axis_index("s") * NC + lax.axis_index("c")
    base = wid * BPW

    pltpu.sync_copy(head_hbm.at[pl.ds(base, BPW)], idx_h)
    pltpu.sync_copy(rel_hbm.at[pl.ds(base, BPW)], idx_r)
    pltpu.sync_copy(tail_hbm.at[pl.ds(base, BPW)], idx_t)

    lanes = lax.iota(jnp.int32, L)

    def fire(chunk, b):
        off = pl.multiple_of(chunk * CH, CH)
        ihv = idx_h[pl.ds(off, CH)]
        irv = idx_r[pl.ds(off, CH)]
        itv = idx_t[pl.ds(off, CH)]
        hbuf, rbuf, tbuf = bufs[b]
        for k in range(CH):
            dst = pl.ds(k, 1)
            pltpu.async_copy(node_hbm.at[pl.ds(ihv[k], 1), :],
                             hbuf.at[dst], sems[b])
            pltpu.async_copy(relemb_hbm.at[pl.ds(irv[k], 1), :],
                             rbuf.at[dst], sems[b])
            pltpu.async_copy(node_hbm.at[pl.ds(itv[k], 1), :],
                             tbuf.at[dst], sems[b])

    def drain_and_compute(chunk, b):
        hbuf, rbuf, tbuf = bufs[b]
        for buf in (hbuf, rbuf, tbuf):
            pltpu.make_async_copy(node_hbm.at[pl.ds(0, CH), :], buf,
                                  sems[b]).wait()

        def col_body(j, acc):
            cj = jnp.full((L,), j, dtype=jnp.int32)
            h = plsc.load_gather(hbuf, [lanes, cj])
            r = plsc.load_gather(rbuf, [lanes, cj])
            t = plsc.load_gather(tbuf, [lanes, cj])
            return acc + jnp.abs(h + r - t)

        acc = lax.fori_loop(0, HIDDEN, col_body, jnp.zeros((L,), jnp.float32))
        scores[pl.ds(pl.multiple_of(chunk * CH, CH), CH)] = -acc

    for b in range(NBUF - 1):
        fire(b, b)

    def ring_body(c, carry):
        for b in range(NBUF):
            nxt = c + b + (NBUF - 1)

            @pl.when(nxt < NCHUNK)
            def _():
                fire(nxt, (b + NBUF - 1) % NBUF)

            drain_and_compute(c + b, b)
        return carry

    lax.fori_loop(0, NCHUNK // NBUF, lambda i, cy: ring_body(i * NBUF, cy), 0)

    pltpu.sync_copy(scores, out_hbm.at[pl.ds(base, BPW)])


def kernel(head_index, rel_type, tail_index, node_emb, rel_emb):
    return _kge_score_sc(
        head_index.astype(jnp.int32),
        rel_type.astype(jnp.int32),
        tail_index.astype(jnp.int32),
        node_emb,
        rel_emb,
    )
